# baseline (speedup 1.0000x reference)
import functools
import jax
import jax.numpy as jnp
from jax import lax
from jax.experimental import pallas as pl
from jax.experimental.pallas import tpu as pltpu
from jax.experimental.pallas import tpu_sc as plsc

_D = 64
_C = 1024
_ISUB = 128


@functools.cache
def _build(B, V, NW):
    b_per_w = B // NW
    n_chunks = b_per_w // _C
    nsub = _C // _ISUB

    info = plsc.get_sparse_core_info()
    nc = info.num_cores
    mesh = plsc.VectorSubcoreMesh(core_axis_name="c", subcore_axis_name="s")

    @functools.partial(
        pl.kernel,
        mesh=mesh,
        out_type=jax.ShapeDtypeStruct((B, _D), jnp.float32),
        scratch_types=[
            pltpu.VMEM((nsub, _ISUB), jnp.int32),
            pltpu.VMEM((_C, _D), jnp.float32),
            pltpu.VMEM((_C,), jnp.float32),
            pltpu.SemaphoreType.DMA,
            pltpu.SemaphoreType.DMA,
        ],
        compiler_params=pltpu.CompilerParams(use_tc_tiling_on_sc=False),
    )
    def gather_kernel(x_hbm, w_hbm, m_hbm, out_hbm, idx_v, rows_v, mv_v,
                      sem_r, sem_m):
        wid = lax.axis_index("s") * nc + lax.axis_index("c")
        base = wid * b_per_w

        def chunk(g, carry):
            cb = pl.multiple_of(base + g * _C, _C)
            row0 = pl.multiple_of(cb // _ISUB, 8)
            pltpu.sync_copy(x_hbm.at[pl.ds(row0, nsub)], idx_v)
            copies = []
            for j in range(nsub):
                copies.append(pltpu.async_copy(
                    w_hbm.at[idx_v.at[j]],
                    rows_v.at[pl.ds(j * _ISUB, _ISUB)], sem_r))
                copies.append(pltpu.async_copy(
                    m_hbm.at[idx_v.at[j]],
                    mv_v.at[pl.ds(j * _ISUB, _ISUB)], sem_m))
            for c in copies:
                c.wait()

            def group(g16, carry2):
                rbase = g16 * 16
                mvec = mv_v[pl.ds(rbase, 16)]
                for i in range(16):
                    mval = mvec[i]
                    for k in range(_D // 16):
                        sl = pl.ds(k * 16, 16)
                        rows_v[rbase + i, sl] = rows_v[rbase + i, sl] * mval
                return carry2

            lax.fori_loop(0, _C // 16, group, 0)

            pltpu.sync_copy(rows_v, out_hbm.at[pl.ds(cb, _C)])
            return carry

        lax.fori_loop(0, n_chunks, chunk, 0)

    return gather_kernel


def kernel(x, W, mask):
    B = x.shape[0] * x.shape[1]
    V = W.shape[0]
    x2 = x.reshape(B // _ISUB, _ISUB)
    m_flat = mask.reshape(V)
    out = _build(B, V, 32)(x2, W, m_flat)
    return out.reshape(x.shape[0], x.shape[1], _D)
